# Initial kernel scaffold; baseline (speedup 1.0000x reference)
import functools

import jax
import jax.numpy as jnp
from jax import lax
from jax.experimental import pallas as pl
from jax.experimental.pallas import tpu as pltpu
from jax.experimental.pallas import tpu_sc as plsc

N = 10000
E = 320000
D = 128

NC = 2
NS = 16
NW = NC * NS
EPT = E // NW
K = 80
NCHUNK = EPT // K
RPT = N // NS
ZR = 125
DW = 16

_mesh = plsc.VectorSubcoreMesh(core_axis_name="c", subcore_axis_name="s")


@functools.partial(
    pl.kernel,
    out_type=(
        jax.ShapeDtypeStruct((NC, N, DW), jnp.float32),
        jax.ShapeDtypeStruct((NC, N, DW), jnp.float32),
    ),
    mesh=_mesh,
    scratch_types=[
        pltpu.VMEM((NCHUNK, K), jnp.int32),
        pltpu.VMEM((NCHUNK, K), jnp.int32),
        pltpu.VMEM((K, DW), jnp.float32),
        pltpu.VMEM((ZR, DW), jnp.float32),
        pltpu.VMEM_SHARED((N, DW), jnp.float32),
        pltpu.VMEM_SHARED((N, DW), jnp.float32),
    ],
)
def _deg_kernel(src_hbm, dst_hbm, dout_hbm, din_hbm,
                src_v, dst_v, ones_v, zb_v, dout_sp, din_sp):
    c = lax.axis_index("c")
    s = lax.axis_index("s")
    wid = c * NS + s

    pltpu.sync_copy(src_hbm.at[pl.ds(wid * NCHUNK, NCHUNK)], src_v)
    pltpu.sync_copy(dst_hbm.at[pl.ds(wid * NCHUNK, NCHUNK)], dst_v)

    ones16 = jnp.ones((16,), jnp.float32)
    zeros16 = jnp.zeros((16,), jnp.float32)

    @pl.loop(0, K)
    def _(r):
        ones_v[r, pl.ds(0, 16)] = ones16

    @pl.loop(0, ZR)
    def _(r):
        zb_v[r, pl.ds(0, 16)] = zeros16

    @pl.loop(0, RPT // ZR)
    def _(k):
        base = s * RPT + k * ZR
        pltpu.sync_copy(zb_v, dout_sp.at[pl.ds(base, ZR)])
        pltpu.sync_copy(zb_v, din_sp.at[pl.ds(base, ZR)])

    plsc.subcore_barrier()

    @pl.loop(0, NCHUNK)
    def _(j):
        pltpu.sync_copy(ones_v, dout_sp.at[src_v.at[j]], add=True)
        pltpu.sync_copy(ones_v, din_sp.at[dst_v.at[j]], add=True)

    plsc.subcore_barrier()

    @pl.loop(0, RPT // ZR)
    def _(k):
        base = s * RPT + k * ZR
        pltpu.sync_copy(dout_sp.at[pl.ds(base, ZR)], zb_v)
        pltpu.sync_copy(zb_v, dout_hbm.at[c, pl.ds(base, ZR)])
        pltpu.sync_copy(din_sp.at[pl.ds(base, ZR)], zb_v)
        pltpu.sync_copy(zb_v, din_hbm.at[c, pl.ds(base, ZR)])


@functools.partial(
    pl.kernel,
    out_type=jax.ShapeDtypeStruct((NC, N, D), jnp.float32),
    mesh=_mesh,
    scratch_types=[
        pltpu.VMEM((EPT,), jnp.int32),
        pltpu.VMEM((NCHUNK, K), jnp.int32),
        pltpu.VMEM((K, D), jnp.float32),
        pltpu.VMEM((ZR, D), jnp.float32),
        pltpu.VMEM_SHARED((N, D), jnp.float32),
        pltpu.SemaphoreType.DMA,
    ],
)
def _agg_kernel(h_hbm, src_hbm, dst_hbm, out_hbm,
                src_v, dst_v, rows_v, zb_v, acc_sp, sem):
    c = lax.axis_index("c")
    s = lax.axis_index("s")
    wid = c * NS + s

    pltpu.sync_copy(src_hbm.at[pl.ds(wid * EPT, EPT)], src_v)
    pltpu.sync_copy(dst_hbm.at[pl.ds(wid * NCHUNK, NCHUNK)], dst_v)

    zeros16 = jnp.zeros((16,), jnp.float32)

    @pl.loop(0, ZR)
    def _(r):
        @pl.loop(0, D // 16)
        def _(l):
            zb_v[r, pl.ds(l * 16, 16)] = zeros16

    @pl.loop(0, RPT // ZR)
    def _(k):
        pltpu.sync_copy(zb_v, acc_sp.at[pl.ds(s * RPT + k * ZR, ZR)])

    plsc.subcore_barrier()

    @pl.loop(0, NCHUNK)
    def _(j):
        pltpu.async_copy(h_hbm.at[src_v.at[pl.ds(j * K, K)]], rows_v, sem).wait()
        pltpu.sync_copy(rows_v, acc_sp.at[dst_v.at[j]], add=True)

    plsc.subcore_barrier()

    @pl.loop(0, RPT // ZR)
    def _(k):
        base = s * RPT + k * ZR
        pltpu.sync_copy(acc_sp.at[pl.ds(base, ZR)], zb_v)
        pltpu.sync_copy(zb_v, out_hbm.at[c, pl.ds(base, ZR)])


BN = 2000
_GRID = N // BN


def _tc_a_body(x_ref, dpo_ref, dpi_ref, xs_ref, nsrc_ref, ndst_ref):
    deg_o = dpo_ref[0, :, 0] + dpo_ref[1, :, 0]
    deg_i = dpi_ref[0, :, 0] + dpi_ref[1, :, 0]
    n_s = jnp.where(deg_o > 0, lax.rsqrt(deg_o), 0.0)
    n_d = jnp.where(deg_i > 0, lax.rsqrt(deg_i), 0.0)
    xs_ref[...] = x_ref[...] * n_s[:, None]
    nsrc_ref[0, :] = n_s
    ndst_ref[0, :] = n_d


def _tc_a(x, dpo, dpi):
    return pl.pallas_call(
        _tc_a_body,
        grid=(_GRID,),
        in_specs=[
            pl.BlockSpec((BN, D), lambda i: (i, 0)),
            pl.BlockSpec((NC, BN, DW), lambda i: (0, i, 0)),
            pl.BlockSpec((NC, BN, DW), lambda i: (0, i, 0)),
        ],
        out_specs=[
            pl.BlockSpec((BN, D), lambda i: (i, 0)),
            pl.BlockSpec((1, BN), lambda i: (i, 0)),
            pl.BlockSpec((1, BN), lambda i: (i, 0)),
        ],
        out_shape=[
            jax.ShapeDtypeStruct((N, D), jnp.float32),
            jax.ShapeDtypeStruct((_GRID, BN), jnp.float32),
            jax.ShapeDtypeStruct((_GRID, BN), jnp.float32),
        ],
    )(x, dpo, dpi)


def _tc_b_body(p_ref, ndst_ref, nsrc_ref, w_ref, b_ref, h1_ref, h1s_ref):
    agg = (p_ref[0] + p_ref[1]) * ndst_ref[0, :][:, None]
    h = jnp.dot(agg, w_ref[...], preferred_element_type=jnp.float32,
                precision=lax.Precision.HIGHEST) + b_ref[...]
    h1 = jnp.maximum(h, 0.0)
    h1_ref[...] = h1
    h1s_ref[...] = h1 * nsrc_ref[0, :][:, None]


def _tc_b(p, ndst, nsrc, w, b):
    return pl.pallas_call(
        _tc_b_body,
        grid=(_GRID,),
        in_specs=[
            pl.BlockSpec((NC, BN, D), lambda i: (0, i, 0)),
            pl.BlockSpec((1, BN), lambda i: (i, 0)),
            pl.BlockSpec((1, BN), lambda i: (i, 0)),
            pl.BlockSpec((D, D), lambda i: (0, 0)),
            pl.BlockSpec((1, D), lambda i: (0, 0)),
        ],
        out_specs=[
            pl.BlockSpec((BN, D), lambda i: (i, 0)),
            pl.BlockSpec((BN, D), lambda i: (i, 0)),
        ],
        out_shape=[
            jax.ShapeDtypeStruct((N, D), jnp.float32),
            jax.ShapeDtypeStruct((N, D), jnp.float32),
        ],
    )(p, ndst, nsrc, w, b)


def _tc_c_body(p_ref, ndst_ref, w_ref, b_ref, h2_ref):
    agg = (p_ref[0] + p_ref[1]) * ndst_ref[0, :][:, None]
    h2_ref[...] = jnp.dot(agg, w_ref[...], preferred_element_type=jnp.float32,
                          precision=lax.Precision.HIGHEST) + b_ref[...]


def _tc_c(p, ndst, w, b):
    return pl.pallas_call(
        _tc_c_body,
        grid=(_GRID,),
        in_specs=[
            pl.BlockSpec((NC, BN, D), lambda i: (0, i, 0)),
            pl.BlockSpec((1, BN), lambda i: (i, 0)),
            pl.BlockSpec((D, D), lambda i: (0, 0)),
            pl.BlockSpec((1, D), lambda i: (0, 0)),
        ],
        out_specs=pl.BlockSpec((BN, D), lambda i: (i, 0)),
        out_shape=jax.ShapeDtypeStruct((N, D), jnp.float32),
    )(p, ndst, w, b)


def kernel(inputs, edge_index, W1, b1, W2, b2):
    src = edge_index[0]
    dst = edge_index[1]
    src2 = src.reshape(E // K, K)
    dst2 = dst.reshape(E // K, K)

    dpo, dpi = _deg_kernel(src2, dst2)
    xs, nsrc, ndst = _tc_a(inputs, dpo, dpi)
    p1 = _agg_kernel(xs, src, dst2)
    h1, h1s = _tc_b(p1, ndst, nsrc, W1, b1.reshape(1, D))
    p2 = _agg_kernel(h1s, src, dst2)
    h2 = _tc_c(p2, ndst, W2, b2.reshape(1, D))
    return (h2, h1, inputs)


# speedup vs baseline: 6.2466x; 6.2466x over previous
import functools

import jax
import jax.numpy as jnp
from jax import lax
from jax.experimental import pallas as pl
from jax.experimental.pallas import tpu as pltpu
from jax.experimental.pallas import tpu_sc as plsc

N = 10000
E = 320000
D = 128

NC = 2
NS = 16
NW = NC * NS
EPT = E // NW
K = 80
NCHUNK = EPT // K
N_PAD = 10240
RPT = N_PAD // NS
ZR = 128
DW = 16

_mesh = plsc.VectorSubcoreMesh(core_axis_name="c", subcore_axis_name="s")


EPT2 = E // NS
NCHUNK2 = EPT2 // K


@functools.partial(
    pl.kernel,
    out_type=jax.ShapeDtypeStruct((NC, N_PAD, D), jnp.float32),
    mesh=_mesh,
    scratch_types=[
        pltpu.VMEM((NCHUNK2, K), jnp.int32),
        pltpu.VMEM((K, D), jnp.float32),
        pltpu.VMEM_SHARED((N_PAD, D), jnp.float32),
    ],
)
def _deg_kernel(ei_hbm, ones_hbm, zrow_hbm, deg_hbm, idx_v, ones_v, deg_sp):
    c = lax.axis_index("c")
    s = lax.axis_index("s")

    pltpu.sync_copy(ei_hbm.at[c, s], idx_v)
    pltpu.sync_copy(ones_hbm, ones_v)

    @pl.loop(0, RPT // K)
    def _(k):
        pltpu.sync_copy(zrow_hbm, deg_sp.at[pl.ds(s * RPT + k * K, K)])

    plsc.subcore_barrier()

    @pl.loop(0, NCHUNK2)
    def _(j):
        pltpu.sync_copy(ones_v, deg_sp.at[idx_v.at[j]], add=True)

    plsc.subcore_barrier()

    @pl.loop(0, RPT // K)
    def _(k):
        base = s * RPT + k * K
        pltpu.sync_copy(deg_sp.at[pl.ds(base, K)], deg_hbm.at[c, pl.ds(base, K)])


@functools.partial(
    pl.kernel,
    out_type=jax.ShapeDtypeStruct((NC, N_PAD, D), jnp.float32),
    mesh=_mesh,
    scratch_types=[
        pltpu.VMEM((NCHUNK, K), jnp.int32),
        pltpu.VMEM((NCHUNK, K), jnp.int32),
        pltpu.VMEM((K, D), jnp.float32),
        pltpu.VMEM_SHARED((N_PAD, D), jnp.float32),
        pltpu.SemaphoreType.DMA,
    ],
)
def _agg_kernel(h_hbm, src_hbm, dst_hbm, zrow_hbm, out_hbm,
                src_v, dst_v, rows_v, acc_sp, sem):
    c = lax.axis_index("c")
    s = lax.axis_index("s")
    wid = c * NS + s

    pltpu.sync_copy(src_hbm.at[wid], src_v)
    pltpu.sync_copy(dst_hbm.at[wid], dst_v)

    @pl.loop(0, RPT // K)
    def _(k):
        pltpu.sync_copy(zrow_hbm, acc_sp.at[pl.ds(s * RPT + k * K, K)])

    plsc.subcore_barrier()

    @pl.loop(0, NCHUNK)
    def _(j):
        pltpu.async_copy(h_hbm.at[src_v.at[j]], rows_v, sem).wait()
        pltpu.sync_copy(rows_v, acc_sp.at[dst_v.at[j]], add=True)

    plsc.subcore_barrier()

    @pl.loop(0, RPT // K)
    def _(k):
        base = s * RPT + k * K
        pltpu.sync_copy(acc_sp.at[pl.ds(base, K)], out_hbm.at[c, pl.ds(base, K)])


BN = 2000
_GRID = N // BN


def _tc_a_body(x_ref, deg_ref, xs_ref, nsrc_ref, ndst_ref):
    deg_o = deg_ref[0, :, 0]
    deg_i = deg_ref[1, :, 0]
    n_s = jnp.where(deg_o > 0, lax.rsqrt(deg_o), 0.0)
    n_d = jnp.where(deg_i > 0, lax.rsqrt(deg_i), 0.0)
    xs_ref[...] = x_ref[...] * n_s[:, None]
    nsrc_ref[0, 0, :] = n_s
    ndst_ref[0, 0, :] = n_d


def _tc_a(x, deg):
    return pl.pallas_call(
        _tc_a_body,
        grid=(_GRID,),
        in_specs=[
            pl.BlockSpec((BN, D), lambda i: (i, 0)),
            pl.BlockSpec((NC, BN, D), lambda i: (0, i, 0)),
        ],
        out_specs=[
            pl.BlockSpec((BN, D), lambda i: (i, 0)),
            pl.BlockSpec((1, 1, BN), lambda i: (i, 0, 0)),
            pl.BlockSpec((1, 1, BN), lambda i: (i, 0, 0)),
        ],
        out_shape=[
            jax.ShapeDtypeStruct((N, D), jnp.float32),
            jax.ShapeDtypeStruct((_GRID, 1, BN), jnp.float32),
            jax.ShapeDtypeStruct((_GRID, 1, BN), jnp.float32),
        ],
    )(x, deg)


def _tc_b_body(p_ref, ndst_ref, nsrc_ref, w_ref, b_ref, h1_ref, h1s_ref):
    agg = (p_ref[0] + p_ref[1]) * ndst_ref[0, 0, :][:, None]
    h = jnp.dot(agg, w_ref[...], preferred_element_type=jnp.float32,
                precision=lax.Precision.HIGHEST) + b_ref[...]
    h1 = jnp.maximum(h, 0.0)
    h1_ref[...] = h1
    h1s_ref[...] = h1 * nsrc_ref[0, 0, :][:, None]


def _tc_b(p, ndst, nsrc, w, b):
    return pl.pallas_call(
        _tc_b_body,
        grid=(_GRID,),
        in_specs=[
            pl.BlockSpec((NC, BN, D), lambda i: (0, i, 0)),
            pl.BlockSpec((1, 1, BN), lambda i: (i, 0, 0)),
            pl.BlockSpec((1, 1, BN), lambda i: (i, 0, 0)),
            pl.BlockSpec((D, D), lambda i: (0, 0)),
            pl.BlockSpec((1, D), lambda i: (0, 0)),
        ],
        out_specs=[
            pl.BlockSpec((BN, D), lambda i: (i, 0)),
            pl.BlockSpec((BN, D), lambda i: (i, 0)),
        ],
        out_shape=[
            jax.ShapeDtypeStruct((N, D), jnp.float32),
            jax.ShapeDtypeStruct((N, D), jnp.float32),
        ],
    )(p, ndst, nsrc, w, b)


def _tc_c_body(p_ref, ndst_ref, w_ref, b_ref, h2_ref):
    agg = (p_ref[0] + p_ref[1]) * ndst_ref[0, 0, :][:, None]
    h2_ref[...] = jnp.dot(agg, w_ref[...], preferred_element_type=jnp.float32,
                          precision=lax.Precision.HIGHEST) + b_ref[...]


def _tc_c(p, ndst, w, b):
    return pl.pallas_call(
        _tc_c_body,
        grid=(_GRID,),
        in_specs=[
            pl.BlockSpec((NC, BN, D), lambda i: (0, i, 0)),
            pl.BlockSpec((1, 1, BN), lambda i: (i, 0, 0)),
            pl.BlockSpec((D, D), lambda i: (0, 0)),
            pl.BlockSpec((1, D), lambda i: (0, 0)),
        ],
        out_specs=pl.BlockSpec((BN, D), lambda i: (i, 0)),
        out_shape=jax.ShapeDtypeStruct((N, D), jnp.float32),
    )(p, ndst, w, b)


def kernel(inputs, edge_index, W1, b1, W2, b2):
    src = edge_index[0]
    dst = edge_index[1]
    src3 = src.reshape(NW, NCHUNK, K)
    dst3 = dst.reshape(NW, NCHUNK, K)
    ei4 = edge_index.reshape(NC, NS, NCHUNK2, K)
    ones_kd = jnp.ones((K, D), jnp.float32)
    zrow = jnp.zeros((K, D), jnp.float32)

    deg = _deg_kernel(ei4, ones_kd, zrow)
    xs, nsrc, ndst = _tc_a(inputs, deg)
    p1 = _agg_kernel(xs, src3, dst3, zrow)
    h1, h1s = _tc_b(p1, ndst, nsrc, W1, b1.reshape(1, D))
    p2 = _agg_kernel(h1s, src3, dst3, zrow)
    h2 = _tc_c(p2, ndst, W2, b2.reshape(1, D))
    return (h2, h1, inputs)


# speedup vs baseline: 11.5564x; 1.8500x over previous
import dataclasses
import functools

import jax
import jax.numpy as jnp
from jax import lax
from jax.experimental import pallas as pl
from jax.experimental.pallas import tpu as pltpu
from jax.experimental.pallas import tpu_sc as plsc

N = 10000
E = 320000
D = 128

NC = 2
NS = 16
NW = NC * NS
EPT = E // NW
K = 80
NCHUNK = EPT // K
N_PAD = 10240
RPT = N_PAD // NS
BN = 2048
_GRID = N_PAD // BN

_mesh = plsc.VectorSubcoreMesh(core_axis_name="c", subcore_axis_name="s")

_cp = pltpu.CompilerParams()
if "needs_layout_passes" in pltpu.CompilerParams.__dataclass_fields__:
    _cp = dataclasses.replace(_cp, needs_layout_passes=False)


@functools.partial(
    pl.kernel,
    out_type=(
        jax.ShapeDtypeStruct((NW, _GRID, 1, BN), jnp.float32),
        jax.ShapeDtypeStruct((NW, _GRID, 1, BN), jnp.float32),
    ),
    mesh=_mesh,
    compiler_params=_cp,
    scratch_types=[
        pltpu.VMEM((EPT,), jnp.int32),
        pltpu.VMEM((EPT,), jnp.int32),
        pltpu.VMEM((_GRID, 1, BN), jnp.float32),
        pltpu.VMEM((_GRID, 1, BN), jnp.float32),
    ],
)
def _deg_kernel(src_hbm, dst_hbm, zh_hbm, dpo_hbm, dpi_hbm,
                src_v, dst_v, ho_v, hi_v):
    c = lax.axis_index("c")
    s = lax.axis_index("s")
    wid = c * NS + s

    pltpu.sync_copy(src_hbm.at[pl.ds(wid * EPT, EPT)], src_v)
    pltpu.sync_copy(dst_hbm.at[pl.ds(wid * EPT, EPT)], dst_v)
    pltpu.sync_copy(zh_hbm, ho_v)
    pltpu.sync_copy(zh_hbm, hi_v)

    ones16 = jnp.ones((16,), jnp.float32)
    zeros16 = jnp.zeros((16,), jnp.int32)

    @pl.loop(0, EPT // 16)
    def _(i):
        sv = src_v[pl.ds(i * 16, 16)]
        plsc.addupdate_scatter(
            ho_v, [sv >> 11, zeros16, sv & (BN - 1)], ones16)
        dv = dst_v[pl.ds(i * 16, 16)]
        plsc.addupdate_scatter(
            hi_v, [dv >> 11, zeros16, dv & (BN - 1)], ones16)

    pltpu.sync_copy(ho_v, dpo_hbm.at[wid])
    pltpu.sync_copy(hi_v, dpi_hbm.at[wid])


@functools.partial(
    pl.kernel,
    out_type=jax.ShapeDtypeStruct((NC, N_PAD, D), jnp.float32),
    mesh=_mesh,
    scratch_types=[
        pltpu.VMEM((EPT,), jnp.int32),
        pltpu.VMEM((NCHUNK, K), jnp.int32),
        pltpu.VMEM((K, D), jnp.float32),
        pltpu.VMEM((K, D), jnp.float32),
        pltpu.VMEM_SHARED((N_PAD, D), jnp.float32),
        pltpu.SemaphoreType.DMA,
        pltpu.SemaphoreType.DMA,
    ],
)
def _agg_kernel(h_hbm, src_hbm, dst_hbm, zrow_hbm, out_hbm,
                src_v, dst_v, buf_a, buf_b, acc_sp, sem_a, sem_b):
    c = lax.axis_index("c")
    s = lax.axis_index("s")
    wid = c * NS + s

    pltpu.sync_copy(src_hbm.at[pl.ds(wid * EPT, EPT)], src_v)
    pltpu.sync_copy(dst_hbm.at[wid], dst_v)

    @pl.loop(0, RPT // K)
    def _(k):
        pltpu.sync_copy(zrow_hbm, acc_sp.at[pl.ds(s * RPT + k * K, K)])

    plsc.subcore_barrier()

    pltpu.async_copy(h_hbm.at[src_v.at[pl.ds(0, K)]], buf_a, sem_a)

    @pl.loop(0, (NCHUNK - 1) // 2)
    def _(jj):
        j0 = 2 * jj
        pltpu.async_copy(h_hbm.at[src_v.at[pl.ds((j0 + 1) * K, K)]],
                         buf_b, sem_b)
        pltpu.make_async_copy(h_hbm.at[src_v.at[pl.ds(j0 * K, K)]],
                              buf_a, sem_a).wait()
        pltpu.sync_copy(buf_a, acc_sp.at[dst_v.at[j0]], add=True)
        pltpu.async_copy(h_hbm.at[src_v.at[pl.ds((j0 + 2) * K, K)]],
                         buf_a, sem_a)
        pltpu.make_async_copy(h_hbm.at[src_v.at[pl.ds((j0 + 1) * K, K)]],
                              buf_b, sem_b).wait()
        pltpu.sync_copy(buf_b, acc_sp.at[dst_v.at[j0 + 1]], add=True)

    pltpu.make_async_copy(h_hbm.at[src_v.at[pl.ds((NCHUNK - 1) * K, K)]],
                          buf_a, sem_a).wait()
    pltpu.sync_copy(buf_a, acc_sp.at[dst_v.at[NCHUNK - 1]], add=True)

    plsc.subcore_barrier()

    @pl.loop(0, RPT // K)
    def _(k):
        base = s * RPT + k * K
        pltpu.sync_copy(acc_sp.at[pl.ds(base, K)],
                        out_hbm.at[c, pl.ds(base, K)])


def _tc_a_body(x_ref, dpo_ref, dpi_ref, xs_ref, nsrc_ref, ndst_ref):
    deg_o = jnp.sum(dpo_ref[:, 0, 0, :], axis=0)
    deg_i = jnp.sum(dpi_ref[:, 0, 0, :], axis=0)
    n_s = jnp.where(deg_o > 0, lax.rsqrt(deg_o), 0.0)
    n_d = jnp.where(deg_i > 0, lax.rsqrt(deg_i), 0.0)
    xs_ref[...] = x_ref[...] * n_s[:, None]
    nsrc_ref[0, 0, :] = n_s
    ndst_ref[0, 0, :] = n_d


def _tc_a(x, dpo, dpi):
    return pl.pallas_call(
        _tc_a_body,
        grid=(_GRID,),
        in_specs=[
            pl.BlockSpec((BN, D), lambda i: (i, 0)),
            pl.BlockSpec((NW, 1, 1, BN), lambda i: (0, i, 0, 0)),
            pl.BlockSpec((NW, 1, 1, BN), lambda i: (0, i, 0, 0)),
        ],
        out_specs=[
            pl.BlockSpec((BN, D), lambda i: (i, 0)),
            pl.BlockSpec((1, 1, BN), lambda i: (i, 0, 0)),
            pl.BlockSpec((1, 1, BN), lambda i: (i, 0, 0)),
        ],
        out_shape=[
            jax.ShapeDtypeStruct((N, D), jnp.float32),
            jax.ShapeDtypeStruct((_GRID, 1, BN), jnp.float32),
            jax.ShapeDtypeStruct((_GRID, 1, BN), jnp.float32),
        ],
    )(x, dpo, dpi)


def _tc_b_body(p_ref, ndst_ref, nsrc_ref, w_ref, b_ref, h1_ref, h1s_ref):
    agg = (p_ref[0] + p_ref[1]) * ndst_ref[0, 0, :][:, None]
    h = jnp.dot(agg, w_ref[...], preferred_element_type=jnp.float32,
                precision=lax.Precision.HIGHEST) + b_ref[...]
    h1 = jnp.maximum(h, 0.0)
    h1_ref[...] = h1
    h1s_ref[...] = h1 * nsrc_ref[0, 0, :][:, None]


def _tc_b(p, ndst, nsrc, w, b):
    return pl.pallas_call(
        _tc_b_body,
        grid=(_GRID,),
        in_specs=[
            pl.BlockSpec((NC, BN, D), lambda i: (0, i, 0)),
            pl.BlockSpec((1, 1, BN), lambda i: (i, 0, 0)),
            pl.BlockSpec((1, 1, BN), lambda i: (i, 0, 0)),
            pl.BlockSpec((D, D), lambda i: (0, 0)),
            pl.BlockSpec((1, D), lambda i: (0, 0)),
        ],
        out_specs=[
            pl.BlockSpec((BN, D), lambda i: (i, 0)),
            pl.BlockSpec((BN, D), lambda i: (i, 0)),
        ],
        out_shape=[
            jax.ShapeDtypeStruct((N, D), jnp.float32),
            jax.ShapeDtypeStruct((N, D), jnp.float32),
        ],
    )(p, ndst, nsrc, w, b)


def _tc_c_body(p_ref, ndst_ref, w_ref, b_ref, h2_ref):
    agg = (p_ref[0] + p_ref[1]) * ndst_ref[0, 0, :][:, None]
    h2_ref[...] = jnp.dot(agg, w_ref[...], preferred_element_type=jnp.float32,
                          precision=lax.Precision.HIGHEST) + b_ref[...]


def _tc_c(p, ndst, w, b):
    return pl.pallas_call(
        _tc_c_body,
        grid=(_GRID,),
        in_specs=[
            pl.BlockSpec((NC, BN, D), lambda i: (0, i, 0)),
            pl.BlockSpec((1, 1, BN), lambda i: (i, 0, 0)),
            pl.BlockSpec((D, D), lambda i: (0, 0)),
            pl.BlockSpec((1, D), lambda i: (0, 0)),
        ],
        out_specs=pl.BlockSpec((BN, D), lambda i: (i, 0)),
        out_shape=jax.ShapeDtypeStruct((N, D), jnp.float32),
    )(p, ndst, w, b)


def kernel(inputs, edge_index, W1, b1, W2, b2):
    src = edge_index[0]
    dst = edge_index[1]
    dst3 = dst.reshape(NW, NCHUNK, K)
    zhist = jnp.zeros((_GRID, 1, BN), jnp.float32)
    zrow = jnp.zeros((K, D), jnp.float32)

    dpo, dpi = _deg_kernel(src, dst, zhist)
    xs, nsrc, ndst = _tc_a(inputs, dpo, dpi)
    p1 = _agg_kernel(xs, src, dst3, zrow)
    h1, h1s = _tc_b(p1, ndst, nsrc, W1, b1.reshape(1, D))
    p2 = _agg_kernel(h1s, src, dst3, zrow)
    h2 = _tc_c(p2, ndst, W2, b2.reshape(1, D))
    return (h2, h1, inputs)
